# Initial kernel scaffold; baseline (speedup 1.0000x reference)
import jax
import jax.numpy as jnp
from jax.experimental import pallas as pl


def kernel(x, edge_index, W1, b1, W2, b2):
    raise NotImplementedError("write your pallas kernel here")


# speedup vs baseline: 28.5633x; 28.5633x over previous
import functools

import jax
import jax.numpy as jnp
from jax import lax
from jax.experimental import pallas as pl
from jax.experimental.pallas import tpu as pltpu
from jax.experimental.pallas import tpu_sc as plsc

N = 10000
E = 320000
D_IN = 128
HID = 16
D_OUT = 128

NC = 2
NS = 16
NW = NC * NS
Q = E // NW
CH = 128
NCH = (Q + CH - 1) // CH
QP = 10240
NCHP = QP // CH
NP = 10240
TRASH = N + 128
RPT = NP // NS

_mesh = plsc.VectorSubcoreMesh(core_axis_name="c", subcore_axis_name="s")


def _deg_body(dst_hbm, ones_hbm, zero_hbm, out_hbm, dst_v, rows_v, accum_sh):
    cid = lax.axis_index("c")
    sid = lax.axis_index("s")
    wid = sid * NC + cid

    @pl.when(sid == 0)
    def _():
        pltpu.sync_copy(zero_hbm, accum_sh)

    plsc.subcore_barrier()
    pltpu.sync_copy(dst_hbm.at[wid], dst_v)
    pltpu.sync_copy(ones_hbm, rows_v)

    def step(j, carry):
        pltpu.sync_copy(rows_v, accum_sh.at[dst_v.at[j]], add=True)
        return carry

    lax.fori_loop(0, NCHP, step, 0)
    plsc.subcore_barrier()
    pltpu.sync_copy(accum_sh.at[pl.ds(sid * RPT, RPT)],
                    out_hbm.at[cid, pl.ds(sid * RPT, RPT)])


def _agg_body(g_hbm, src_hbm, dst_hbm, zero_hbm, out_hbm,
              src_v, dst_v, rows_v, accum_sh, sem):
    cid = lax.axis_index("c")
    sid = lax.axis_index("s")
    wid = sid * NC + cid

    @pl.when(sid == 0)
    def _():
        pltpu.sync_copy(zero_hbm, accum_sh)

    plsc.subcore_barrier()
    pltpu.sync_copy(src_hbm.at[wid], src_v)
    pltpu.sync_copy(dst_hbm.at[wid], dst_v)

    def step(j, carry):
        pltpu.async_copy(g_hbm.at[src_v.at[j]], rows_v, sem).wait()
        pltpu.sync_copy(rows_v, accum_sh.at[dst_v.at[j]], add=True)
        return carry

    lax.fori_loop(0, NCHP, step, 0)
    plsc.subcore_barrier()
    pltpu.sync_copy(accum_sh.at[pl.ds(sid * RPT, RPT)],
                    out_hbm.at[cid, pl.ds(sid * RPT, RPT)])


_sc_params = pltpu.CompilerParams(use_tc_tiling_on_sc=False)

_deg_call = pl.kernel(
    _deg_body,
    out_type=jax.ShapeDtypeStruct((NC, NP, HID), jnp.float32),
    mesh=_mesh,
    compiler_params=_sc_params,
    scratch_types=[
        pltpu.VMEM((NCHP, CH), jnp.int32),
        pltpu.VMEM((CH, HID), jnp.float32),
        pltpu.VMEM_SHARED((NP, HID), jnp.float32),
    ],
)

_agg_call = pl.kernel(
    _agg_body,
    out_type=jax.ShapeDtypeStruct((NC, NP, HID), jnp.float32),
    mesh=_mesh,
    compiler_params=_sc_params,
    scratch_types=[
        pltpu.VMEM((NCHP, CH), jnp.int32),
        pltpu.VMEM((NCHP, CH), jnp.int32),
        pltpu.VMEM((CH, HID), jnp.float32),
        pltpu.VMEM_SHARED((NP, HID), jnp.float32),
        pltpu.SemaphoreType.DMA,
    ],
)


def _mid1_body(degp_ref, x_ref, w1_ref, g1_ref, dinv_ref):
    deg = degp_ref[0] + degp_ref[1] + 1.0
    dinv = lax.rsqrt(deg)
    h = jnp.dot(x_ref[...], w1_ref[...], preferred_element_type=jnp.float32)
    dinv_ref[...] = dinv
    g1_ref[...] = dinv * h


def _mid2_body(p_ref, g1_ref, dinv_ref, b1_ref, g2_ref):
    s = p_ref[0] + p_ref[1] + g1_ref[...]
    out1 = jnp.maximum(dinv_ref[...] * s + b1_ref[...], 0.0)
    g2_ref[...] = dinv_ref[...] * out1


def _fin_body(q_ref, g2_ref, dinv_ref, w2_ref, b2_ref, out_ref):
    z = dinv_ref[...] * (q_ref[0] + q_ref[1] + g2_ref[...])
    out_ref[...] = (
        jnp.dot(z, w2_ref[...], preferred_element_type=jnp.float32)
        + b2_ref[...]
    )


_mid1_call = pl.pallas_call(
    _mid1_body,
    out_shape=[
        jax.ShapeDtypeStruct((NP, HID), jnp.float32),
        jax.ShapeDtypeStruct((NP, HID), jnp.float32),
    ],
)

_mid2_call = pl.pallas_call(
    _mid2_body,
    out_shape=jax.ShapeDtypeStruct((NP, HID), jnp.float32),
)

_fin_call = pl.pallas_call(
    _fin_body,
    out_shape=jax.ShapeDtypeStruct((NP, D_OUT), jnp.float32),
)


def kernel(x, edge_index, W1, b1, W2, b2):
    src = edge_index[0].astype(jnp.int32).reshape(NW, Q)
    dst = edge_index[1].astype(jnp.int32).reshape(NW, Q)
    src3 = jnp.concatenate(
        [src, jnp.zeros((NW, QP - Q), jnp.int32)], axis=1
    ).reshape(NW, NCHP, CH)
    dst3 = jnp.concatenate(
        [dst, jnp.full((NW, QP - Q), TRASH, jnp.int32)], axis=1
    ).reshape(NW, NCHP, CH)

    ones_rows = jnp.ones((CH, HID), jnp.float32)
    zeros_np = jnp.zeros((NP, HID), jnp.float32)
    x_pad = jnp.concatenate(
        [x, jnp.zeros((NP - N, D_IN), jnp.float32)], axis=0
    )

    degp = _deg_call(dst3, ones_rows, zeros_np)
    g1, dinv = _mid1_call(degp, x_pad, W1)
    p = _agg_call(g1, src3, dst3, zeros_np)
    g2 = _mid2_call(p, g1, dinv, b1.reshape(1, HID))
    q = _agg_call(g2, src3, dst3, zeros_np)
    out = _fin_call(q, g2, dinv, W2, b2.reshape(1, D_OUT))
    return out[:N]
